# baseline (speedup 1.0000x reference)
import jax
import jax.numpy as jnp
from jax import lax
from jax.experimental import pallas as pl
from jax.experimental.pallas import tpu as pltpu
from jax.experimental.pallas import tpu_sc as plsc

N0 = 20000
N1 = 20000
E = 640000
D = 64
H = 32
C = 10
CP = 16

NC = 2
NS = 16
NPAD = 20480
ROWS_PER_TILE = NPAD // NS
K = 80

_ROW_BLK = 2000


def _tc1_body(x0_ref, x1_ref, wr0_ref, wo0_ref, wr1_ref, wo1_ref,
              b0_ref, b1_ref, z0_ref, z1_ref, r1_ref, r0_ref):
    x0 = x0_ref[...]
    x1 = x1_ref[...]
    z0_ref[...] = jnp.dot(x0, wr0_ref[...],
                          preferred_element_type=jnp.float32).astype(jnp.bfloat16)
    z1_ref[...] = jnp.dot(x1, wr1_ref[...],
                          preferred_element_type=jnp.float32).astype(jnp.bfloat16)
    r1_ref[...] = jnp.dot(x1, wo0_ref[...],
                          preferred_element_type=jnp.float32) + b0_ref[...]
    r0_ref[...] = jnp.dot(x0, wo1_ref[...],
                          preferred_element_type=jnp.float32) + b1_ref[...]


def _tc1(x0, x1, wr0, wo0, wr1, wo1, b0, b1):
    g = N0 // _ROW_BLK
    row = pl.BlockSpec((_ROW_BLK, D), lambda i: (i, 0))
    w = pl.BlockSpec((D, H), lambda i: (0, 0))
    b = pl.BlockSpec((1, H), lambda i: (0, 0))
    out = pl.BlockSpec((_ROW_BLK, H), lambda i: (i, 0))
    return pl.pallas_call(
        _tc1_body,
        grid=(g,),
        in_specs=[row, row, w, w, w, w, b, b],
        out_specs=[out, out, out, out],
        out_shape=[jax.ShapeDtypeStruct((N0, H), jnp.bfloat16),
                   jax.ShapeDtypeStruct((N0, H), jnp.bfloat16),
                   jax.ShapeDtypeStruct((N0, H), jnp.float32),
                   jax.ShapeDtypeStruct((N0, H), jnp.float32)],
    )(x0, x1, wr0, wo0, wr1, wo1, b0, b1)


def _tc2_body(a1_ref, r1_ref, a0_ref, r0_ref, wrel_ref, wroot_ref, b2_ref,
              u1_ref, root2_ref):
    h_v1 = jnp.maximum(a1_ref[...].astype(jnp.float32) + r1_ref[...], 0.0)
    h_v0 = jnp.maximum(a0_ref[...].astype(jnp.float32) + r0_ref[...], 0.0)
    pad = ((0, 0), (0, CP - C))
    wrel = jnp.pad(wrel_ref[...], pad)
    wroot = jnp.pad(wroot_ref[...], pad)
    b2 = jnp.pad(b2_ref[...], pad)
    u1_ref[...] = jnp.dot(h_v1, wrel,
                          preferred_element_type=jnp.float32).astype(jnp.bfloat16)
    root2_ref[...] = jnp.dot(h_v0, wroot,
                             preferred_element_type=jnp.float32) + b2


def _tc2(a1, r1, a0, r0, wrel, wroot, b2):
    g = N0 // _ROW_BLK
    rowb = pl.BlockSpec((_ROW_BLK, H), lambda i: (i, 0))
    w = pl.BlockSpec((H, C), lambda i: (0, 0))
    b = pl.BlockSpec((1, C), lambda i: (0, 0))
    out = pl.BlockSpec((_ROW_BLK, CP), lambda i: (i, 0))
    return pl.pallas_call(
        _tc2_body,
        grid=(g,),
        in_specs=[rowb, rowb, rowb, rowb, w, w, b],
        out_specs=[out, out],
        out_shape=[jax.ShapeDtypeStruct((N0, CP), jnp.bfloat16),
                   jax.ShapeDtypeStruct((N0, CP), jnp.float32)],
    )(a1, r1, a0, r0, wrel, wroot, b2)


def _tc3_body(p0_ref, p1_ref, root_ref, y_ref, loss_ref):
    i = pl.program_id(0)
    x = (p0_ref[...] + p1_ref[...]).astype(jnp.float32) + root_ref[...]
    col = lax.broadcasted_iota(jnp.int32, (_ROW_BLK, CP), 1)
    valid = col < C
    xm = jnp.where(valid, x, -1e30)
    m = jnp.max(xm, axis=1, keepdims=True)
    ex = jnp.where(valid, jnp.exp(x - m), 0.0)
    lse = m[:, 0] + jnp.log(jnp.sum(ex, axis=1))
    pick = jnp.sum(jnp.where(col == y_ref[...], x, 0.0), axis=1)
    part = (jnp.sum(lse - pick) * (1.0 / N0)).reshape(1, 1)

    @pl.when(i == 0)
    def _():
        loss_ref[...] = jnp.zeros_like(loss_ref)

    loss_ref[...] += part


def _tc3(p0, p1, root2, y2d):
    g = N0 // _ROW_BLK
    full = pl.BlockSpec((_ROW_BLK, CP), lambda i: (i, 0))
    return pl.pallas_call(
        _tc3_body,
        grid=(g,),
        in_specs=[full, full, full, pl.BlockSpec((_ROW_BLK, 1), lambda i: (i, 0))],
        out_specs=pl.BlockSpec((1, 1), lambda i: (0, 0)),
        out_shape=jax.ShapeDtypeStruct((1, 1), jnp.float32),
    )(p0, p1, root2, y2d)


def _edge_pipeline(z_hbm, idx_src, idx_dst, acc, bufs, gsem, ssem, nch):
    nb = len(bufs)
    for b in range(3):
        pltpu.async_copy(z_hbm.at[idx_src.at[b]], bufs[b], gsem)

    @pl.loop(0, nch, step=nb)
    def _slots(g):
        for b in range(nb):
            ch = g + b
            buf = bufs[b]
            pltpu.make_async_copy(z_hbm.at[idx_src.at[ch]], buf, gsem).wait()
            pltpu.async_copy(buf, acc.at[idx_dst.at[ch]], ssem, add=True)

            @pl.when(ch >= 2)
            def _():
                pltpu.make_async_copy(bufs[(b - 2) % nb],
                                      acc.at[idx_dst.at[ch]], ssem).wait()

            @pl.when(ch + 3 < nch)
            def _():
                pltpu.async_copy(z_hbm.at[idx_src.at[ch + 3]],
                                 bufs[(b + 3) % nb], gsem)

    for b in range(2):
        pltpu.make_async_copy(bufs[b], acc.at[idx_dst.at[0]], ssem).wait()


_NCH1 = E // NS // K


def _sc1_body(z0_hbm, ei0_hbm, z1_hbm, ei1_hbm,
              zeros_hbm, out0_hbm, out1_hbm,
              acc, idx_src, idx_dst, b0, b1, b2, b3, b4, gsem, ssem):
    bufs = (b0, b1, b2, b3, b4)
    c = lax.axis_index("c")
    s = lax.axis_index("s")
    pltpu.sync_copy(zeros_hbm, acc.at[pl.ds(s * ROWS_PER_TILE, ROWS_PER_TILE)])
    plsc.subcore_barrier()

    for p in range(2):
        @pl.when(c == 0)
        def _():
            pltpu.sync_copy(ei0_hbm.at[0, s, p], idx_src)
            pltpu.sync_copy(ei0_hbm.at[1, s, p], idx_dst)
            _edge_pipeline(z0_hbm, idx_src, idx_dst, acc, bufs, gsem, ssem,
                           _NCH1 // 2)

        @pl.when(c == 1)
        def _():
            pltpu.sync_copy(ei1_hbm.at[0, s, p], idx_src)
            pltpu.sync_copy(ei1_hbm.at[1, s, p], idx_dst)
            _edge_pipeline(z1_hbm, idx_src, idx_dst, acc, bufs, gsem, ssem,
                           _NCH1 // 2)

    plsc.subcore_barrier()
    rows = pl.ds(s * ROWS_PER_TILE, ROWS_PER_TILE)

    @pl.when(c == 0)
    def _():
        pltpu.sync_copy(acc.at[rows], out0_hbm.at[rows])

    @pl.when(c == 1)
    def _():
        pltpu.sync_copy(acc.at[rows], out1_hbm.at[rows])


def _sc1(z0, ei0, z1, ei1, zeros_h):
    mesh = plsc.VectorSubcoreMesh(core_axis_name="c", subcore_axis_name="s")
    return pl.kernel(
        _sc1_body,
        compiler_params=pltpu.CompilerParams(use_tc_tiling_on_sc=False),
        out_type=[jax.ShapeDtypeStruct((NPAD, H), jnp.bfloat16),
                  jax.ShapeDtypeStruct((NPAD, H), jnp.bfloat16)],
        mesh=mesh,
        scratch_types=[
            pltpu.VMEM_SHARED((NPAD, H), jnp.bfloat16),
            pltpu.VMEM((_NCH1 // 2, K), jnp.int32),
            pltpu.VMEM((_NCH1 // 2, K), jnp.int32),
            pltpu.VMEM((K, H), jnp.bfloat16),
            pltpu.VMEM((K, H), jnp.bfloat16),
            pltpu.VMEM((K, H), jnp.bfloat16),
            pltpu.VMEM((K, H), jnp.bfloat16),
            pltpu.VMEM((K, H), jnp.bfloat16),
            pltpu.SemaphoreType.DMA,
            pltpu.SemaphoreType.DMA,
        ],
    )(z0, ei0, z1, ei1, zeros_h)


_NCH2 = E // (NC * NS) // K


def _sc2_body(u1_hbm, ei_hbm, zeros_hbm, out0_hbm, out1_hbm,
              acc, idx_src, idx_dst, b0, b1, b2, b3, b4, gsem, ssem):
    bufs = (b0, b1, b2, b3, b4)
    c = lax.axis_index("c")
    s = lax.axis_index("s")
    pltpu.sync_copy(zeros_hbm, acc.at[pl.ds(s * ROWS_PER_TILE, ROWS_PER_TILE)])
    plsc.subcore_barrier()

    for p in range(2):
        pltpu.sync_copy(ei_hbm.at[0, c, s, p], idx_src)
        pltpu.sync_copy(ei_hbm.at[1, c, s, p], idx_dst)
        _edge_pipeline(u1_hbm, idx_src, idx_dst, acc, bufs, gsem, ssem,
                       _NCH2 // 2)

    plsc.subcore_barrier()
    rows = pl.ds(s * ROWS_PER_TILE, ROWS_PER_TILE)

    @pl.when(c == 0)
    def _():
        pltpu.sync_copy(acc.at[rows], out0_hbm.at[rows])

    @pl.when(c == 1)
    def _():
        pltpu.sync_copy(acc.at[rows], out1_hbm.at[rows])


def _sc2(u1, ei1, zeros_h):
    mesh = plsc.VectorSubcoreMesh(core_axis_name="c", subcore_axis_name="s")
    return pl.kernel(
        _sc2_body,
        compiler_params=pltpu.CompilerParams(use_tc_tiling_on_sc=False),
        out_type=[jax.ShapeDtypeStruct((NPAD, CP), jnp.bfloat16),
                  jax.ShapeDtypeStruct((NPAD, CP), jnp.bfloat16)],
        mesh=mesh,
        scratch_types=[
            pltpu.VMEM_SHARED((NPAD, CP), jnp.bfloat16),
            pltpu.VMEM((_NCH2 // 2, K), jnp.int32),
            pltpu.VMEM((_NCH2 // 2, K), jnp.int32),
            pltpu.VMEM((K, CP), jnp.bfloat16),
            pltpu.VMEM((K, CP), jnp.bfloat16),
            pltpu.VMEM((K, CP), jnp.bfloat16),
            pltpu.VMEM((K, CP), jnp.bfloat16),
            pltpu.VMEM((K, CP), jnp.bfloat16),
            pltpu.SemaphoreType.DMA,
            pltpu.SemaphoreType.DMA,
        ],
    )(u1, ei1, zeros_h)


def kernel(x0, x1, edge_index_0, edge_index_1, y,
           W1_rel_0, b1_0, W1_root_0,
           W1_rel_1, b1_1, W1_root_1,
           W2_rel_0, b2_0, W2_root_0,
           W2_rel_1, b2_1, W2_root_1):
    ei0 = edge_index_0.astype(jnp.int32).reshape(2, NS, 2, _NCH1 // 2, K)
    ei1 = edge_index_1.astype(jnp.int32)
    ei1_l1 = ei1.reshape(2, NS, 2, _NCH1 // 2, K)
    ei1_l2 = ei1.reshape(2, NC, NS, 2, _NCH2 // 2, K)

    z0, z1, r1, r0 = _tc1(x0, x1, W1_rel_0, W1_root_0, W1_rel_1, W1_root_1,
                          b1_0.reshape(1, H), b1_1.reshape(1, H))

    zeros32 = jnp.zeros((ROWS_PER_TILE, H), jnp.bfloat16)
    agg_v1, agg_v0 = _sc1(z0, ei0, z1, ei1_l1, zeros32)

    u1, root2 = _tc2(agg_v1, r1, agg_v0, r0, W2_rel_1, W2_root_1,
                     b2_1.reshape(1, C))

    zeros16 = jnp.zeros((ROWS_PER_TILE, CP), jnp.bfloat16)
    p0, p1 = _sc2(u1, ei1_l2, zeros16)

    y2d = y.astype(jnp.int32).reshape(N0, 1)
    loss = _tc3(p0, p1, root2, y2d)
    return loss[0, 0]


# speedup vs baseline: 1.1301x; 1.1301x over previous
import jax
import jax.numpy as jnp
from jax import lax
from jax.experimental import pallas as pl
from jax.experimental.pallas import tpu as pltpu
from jax.experimental.pallas import tpu_sc as plsc

N0 = 20000
N1 = 20000
E = 640000
D = 64
H = 32
C = 10
CP = 16

NC = 2
NS = 16
NPAD = 20480
ROWS_PER_TILE = NPAD // NS
K = 80

_ROW_BLK = 2000


def _tc1_body(x0_ref, x1_ref, wr0_ref, wo0_ref, wr1_ref, wo1_ref,
              b0_ref, b1_ref, z0_ref, z1_ref, r1_ref, r0_ref):
    x0 = x0_ref[...]
    x1 = x1_ref[...]
    z0_ref[...] = jnp.dot(x0, wr0_ref[...], preferred_element_type=jnp.float32)
    z1_ref[...] = jnp.dot(x1, wr1_ref[...], preferred_element_type=jnp.float32)
    r1_ref[...] = jnp.dot(x1, wo0_ref[...],
                          preferred_element_type=jnp.float32) + b0_ref[...]
    r0_ref[...] = jnp.dot(x0, wo1_ref[...],
                          preferred_element_type=jnp.float32) + b1_ref[...]


def _tc1(x0, x1, wr0, wo0, wr1, wo1, b0, b1):
    g = N0 // _ROW_BLK
    row = pl.BlockSpec((_ROW_BLK, D), lambda i: (i, 0))
    w = pl.BlockSpec((D, H), lambda i: (0, 0))
    b = pl.BlockSpec((1, H), lambda i: (0, 0))
    out = pl.BlockSpec((_ROW_BLK, H), lambda i: (i, 0))
    return pl.pallas_call(
        _tc1_body,
        grid=(g,),
        in_specs=[row, row, w, w, w, w, b, b],
        out_specs=[out, out, out, out],
        out_shape=[jax.ShapeDtypeStruct((N0, H), jnp.float32)] * 4,
    )(x0, x1, wr0, wo0, wr1, wo1, b0, b1)


def _tc2_body(a1_ref, r1_ref, a0_ref, r0_ref, wrel_ref, wroot_ref, b2_ref,
              u1_ref, root2_ref):
    h_v1 = jnp.maximum(a1_ref[...] + r1_ref[...], 0.0)
    h_v0 = jnp.maximum(a0_ref[...] + r0_ref[...], 0.0)
    pad = ((0, 0), (0, CP - C))
    wrel = jnp.pad(wrel_ref[...], pad)
    wroot = jnp.pad(wroot_ref[...], pad)
    b2 = jnp.pad(b2_ref[...], pad)
    u1_ref[...] = jnp.dot(h_v1, wrel,
                          preferred_element_type=jnp.float32).astype(jnp.bfloat16)
    root2_ref[...] = jnp.dot(h_v0, wroot,
                             preferred_element_type=jnp.float32) + b2


def _tc2(a1, r1, a0, r0, wrel, wroot, b2):
    g = N0 // _ROW_BLK
    rowb = pl.BlockSpec((_ROW_BLK, H), lambda i: (i, 0))
    w = pl.BlockSpec((H, C), lambda i: (0, 0))
    b = pl.BlockSpec((1, C), lambda i: (0, 0))
    out = pl.BlockSpec((_ROW_BLK, CP), lambda i: (i, 0))
    return pl.pallas_call(
        _tc2_body,
        grid=(g,),
        in_specs=[rowb, rowb, rowb, rowb, w, w, b],
        out_specs=[out, out],
        out_shape=[jax.ShapeDtypeStruct((N0, CP), jnp.bfloat16),
                   jax.ShapeDtypeStruct((N0, CP), jnp.float32)],
    )(a1, r1, a0, r0, wrel, wroot, b2)


def _tc3_body(p0_ref, p1_ref, root_ref, y_ref, loss_ref):
    i = pl.program_id(0)
    x = (p0_ref[...] + p1_ref[...]).astype(jnp.float32) + root_ref[...]
    col = lax.broadcasted_iota(jnp.int32, (_ROW_BLK, CP), 1)
    valid = col < C
    xm = jnp.where(valid, x, -1e30)
    m = jnp.max(xm, axis=1, keepdims=True)
    ex = jnp.where(valid, jnp.exp(x - m), 0.0)
    lse = m[:, 0] + jnp.log(jnp.sum(ex, axis=1))
    pick = jnp.sum(jnp.where(col == y_ref[...], x, 0.0), axis=1)
    part = (jnp.sum(lse - pick) * (1.0 / N0)).reshape(1, 1)

    @pl.when(i == 0)
    def _():
        loss_ref[...] = jnp.zeros_like(loss_ref)

    loss_ref[...] += part


def _tc3(p0, p1, root2, y2d):
    g = N0 // _ROW_BLK
    full = pl.BlockSpec((_ROW_BLK, CP), lambda i: (i, 0))
    return pl.pallas_call(
        _tc3_body,
        grid=(g,),
        in_specs=[full, full, full, pl.BlockSpec((_ROW_BLK, 1), lambda i: (i, 0))],
        out_specs=pl.BlockSpec((1, 1), lambda i: (0, 0)),
        out_shape=jax.ShapeDtypeStruct((1, 1), jnp.float32),
    )(p0, p1, root2, y2d)


def _edge_pipeline(z_hbm, idx_src, idx_dst, acc, bufs, gsem, ssem, nch):
    nb = len(bufs)
    for b in range(3):
        pltpu.async_copy(z_hbm.at[idx_src.at[b]], bufs[b], gsem)

    @pl.loop(0, nch, step=nb)
    def _slots(g):
        for b in range(nb):
            ch = g + b
            buf = bufs[b]
            pltpu.make_async_copy(z_hbm.at[idx_src.at[ch]], buf, gsem).wait()
            pltpu.async_copy(buf, acc.at[idx_dst.at[ch]], ssem, add=True)

            @pl.when(ch >= 2)
            def _():
                pltpu.make_async_copy(bufs[(b - 2) % nb],
                                      acc.at[idx_dst.at[ch]], ssem).wait()

            @pl.when(ch + 3 < nch)
            def _():
                pltpu.async_copy(z_hbm.at[idx_src.at[ch + 3]],
                                 bufs[(b + 3) % nb], gsem)

    for b in range(2):
        pltpu.make_async_copy(bufs[b], acc.at[idx_dst.at[0]], ssem).wait()


_NCH1 = E // NS // K


def _sc1_body(z0_hbm, ei0_hbm, z1_hbm, ei1_hbm,
              zeros_hbm, out0_hbm, out1_hbm,
              acc, idx_src, idx_dst, b0, b1, b2, b3, b4, gsem, ssem):
    bufs = (b0, b1, b2, b3, b4)
    c = lax.axis_index("c")
    s = lax.axis_index("s")
    pltpu.sync_copy(zeros_hbm, acc.at[pl.ds(s * ROWS_PER_TILE, ROWS_PER_TILE)])
    plsc.subcore_barrier()

    for p in range(2):
        @pl.when(c == 0)
        def _():
            pltpu.sync_copy(ei0_hbm.at[0, s, p], idx_src)
            pltpu.sync_copy(ei0_hbm.at[1, s, p], idx_dst)
            _edge_pipeline(z0_hbm, idx_src, idx_dst, acc, bufs, gsem, ssem,
                           _NCH1 // 2)

        @pl.when(c == 1)
        def _():
            pltpu.sync_copy(ei1_hbm.at[0, s, p], idx_src)
            pltpu.sync_copy(ei1_hbm.at[1, s, p], idx_dst)
            _edge_pipeline(z1_hbm, idx_src, idx_dst, acc, bufs, gsem, ssem,
                           _NCH1 // 2)

    plsc.subcore_barrier()
    rows = pl.ds(s * ROWS_PER_TILE, ROWS_PER_TILE)

    @pl.when(c == 0)
    def _():
        pltpu.sync_copy(acc.at[rows], out0_hbm.at[rows])

    @pl.when(c == 1)
    def _():
        pltpu.sync_copy(acc.at[rows], out1_hbm.at[rows])


def _sc1(z0, ei0, z1, ei1, zeros_h):
    mesh = plsc.VectorSubcoreMesh(core_axis_name="c", subcore_axis_name="s")
    return pl.kernel(
        _sc1_body,
        compiler_params=pltpu.CompilerParams(use_tc_tiling_on_sc=False),
        out_type=[jax.ShapeDtypeStruct((NPAD, H), jnp.float32),
                  jax.ShapeDtypeStruct((NPAD, H), jnp.float32)],
        mesh=mesh,
        scratch_types=[
            pltpu.VMEM_SHARED((NPAD, H), jnp.float32),
            pltpu.VMEM((_NCH1 // 2, K), jnp.int32),
            pltpu.VMEM((_NCH1 // 2, K), jnp.int32),
            pltpu.VMEM((K, H), jnp.float32),
            pltpu.VMEM((K, H), jnp.float32),
            pltpu.VMEM((K, H), jnp.float32),
            pltpu.VMEM((K, H), jnp.float32),
            pltpu.VMEM((K, H), jnp.float32),
            pltpu.SemaphoreType.DMA,
            pltpu.SemaphoreType.DMA,
        ],
    )(z0, ei0, z1, ei1, zeros_h)


_NCH2 = E // (NC * NS) // K


def _sc2_body(u1_hbm, ei_hbm, zeros_hbm, out0_hbm, out1_hbm,
              acc, idx_src, idx_dst, b0, b1, b2, b3, b4, gsem, ssem):
    bufs = (b0, b1, b2, b3, b4)
    c = lax.axis_index("c")
    s = lax.axis_index("s")
    pltpu.sync_copy(zeros_hbm, acc.at[pl.ds(s * ROWS_PER_TILE, ROWS_PER_TILE)])
    plsc.subcore_barrier()

    for p in range(2):
        pltpu.sync_copy(ei_hbm.at[0, c, s, p], idx_src)
        pltpu.sync_copy(ei_hbm.at[1, c, s, p], idx_dst)
        _edge_pipeline(u1_hbm, idx_src, idx_dst, acc, bufs, gsem, ssem,
                       _NCH2 // 2)

    plsc.subcore_barrier()
    rows = pl.ds(s * ROWS_PER_TILE, ROWS_PER_TILE)

    @pl.when(c == 0)
    def _():
        pltpu.sync_copy(acc.at[rows], out0_hbm.at[rows])

    @pl.when(c == 1)
    def _():
        pltpu.sync_copy(acc.at[rows], out1_hbm.at[rows])


def _sc2(u1, ei1, zeros_h):
    mesh = plsc.VectorSubcoreMesh(core_axis_name="c", subcore_axis_name="s")
    return pl.kernel(
        _sc2_body,
        compiler_params=pltpu.CompilerParams(use_tc_tiling_on_sc=False),
        out_type=[jax.ShapeDtypeStruct((NPAD, CP), jnp.bfloat16),
                  jax.ShapeDtypeStruct((NPAD, CP), jnp.bfloat16)],
        mesh=mesh,
        scratch_types=[
            pltpu.VMEM_SHARED((NPAD, CP), jnp.bfloat16),
            pltpu.VMEM((_NCH2 // 2, K), jnp.int32),
            pltpu.VMEM((_NCH2 // 2, K), jnp.int32),
            pltpu.VMEM((K, CP), jnp.bfloat16),
            pltpu.VMEM((K, CP), jnp.bfloat16),
            pltpu.VMEM((K, CP), jnp.bfloat16),
            pltpu.VMEM((K, CP), jnp.bfloat16),
            pltpu.VMEM((K, CP), jnp.bfloat16),
            pltpu.SemaphoreType.DMA,
            pltpu.SemaphoreType.DMA,
        ],
    )(u1, ei1, zeros_h)


def kernel(x0, x1, edge_index_0, edge_index_1, y,
           W1_rel_0, b1_0, W1_root_0,
           W1_rel_1, b1_1, W1_root_1,
           W2_rel_0, b2_0, W2_root_0,
           W2_rel_1, b2_1, W2_root_1):
    ei0 = edge_index_0.astype(jnp.int32).reshape(2, NS, 2, _NCH1 // 2, K)
    ei1 = edge_index_1.astype(jnp.int32)
    ei1_l1 = ei1.reshape(2, NS, 2, _NCH1 // 2, K)
    ei1_l2 = ei1.reshape(2, NC, NS, 2, _NCH2 // 2, K)

    z0, z1, r1, r0 = _tc1(x0, x1, W1_rel_0, W1_root_0, W1_rel_1, W1_root_1,
                          b1_0.reshape(1, H), b1_1.reshape(1, H))

    zeros32 = jnp.zeros((ROWS_PER_TILE, H), jnp.float32)
    agg_v1, agg_v0 = _sc1(z0, ei0, z1, ei1_l1, zeros32)

    u1, root2 = _tc2(agg_v1, r1, agg_v0, r0, W2_rel_1, W2_root_1,
                     b2_1.reshape(1, C))

    zeros16 = jnp.zeros((ROWS_PER_TILE, CP), jnp.bfloat16)
    p0, p1 = _sc2(u1, ei1_l2, zeros16)

    y2d = y.astype(jnp.int32).reshape(N0, 1)
    loss = _tc3(p0, p1, root2, y2d)
    return loss[0, 0]


# speedup vs baseline: 1.2811x; 1.1336x over previous
import jax
import jax.numpy as jnp
from jax import lax
from jax.experimental import pallas as pl
from jax.experimental.pallas import tpu as pltpu
from jax.experimental.pallas import tpu_sc as plsc

N0 = 20000
N1 = 20000
E = 640000
D = 64
H = 32
C = 10
CP = 16

NC = 2
NS = 16
NPAD = 20480
ROWS_PER_TILE = NPAD // NS
K = 80

_ROW_BLK = 2000


def _tc1_body(x0_ref, x1_ref, wr0_ref, wo0_ref, wr1_ref, wo1_ref,
              b0_ref, b1_ref, z0_ref, z1_ref, r1_ref, r0_ref):
    x0 = x0_ref[...]
    x1 = x1_ref[...]
    z0_ref[...] = jnp.dot(x0, wr0_ref[...], preferred_element_type=jnp.float32)
    z1_ref[...] = jnp.dot(x1, wr1_ref[...], preferred_element_type=jnp.float32)
    r1_ref[...] = jnp.dot(x1, wo0_ref[...],
                          preferred_element_type=jnp.float32) + b0_ref[...]
    r0_ref[...] = jnp.dot(x0, wo1_ref[...],
                          preferred_element_type=jnp.float32) + b1_ref[...]


def _tc1(x0, x1, wr0, wo0, wr1, wo1, b0, b1):
    g = N0 // _ROW_BLK
    row = pl.BlockSpec((_ROW_BLK, D), lambda i: (i, 0))
    w = pl.BlockSpec((D, H), lambda i: (0, 0))
    b = pl.BlockSpec((1, H), lambda i: (0, 0))
    out = pl.BlockSpec((_ROW_BLK, H), lambda i: (i, 0))
    return pl.pallas_call(
        _tc1_body,
        grid=(g,),
        in_specs=[row, row, w, w, w, w, b, b],
        out_specs=[out, out, out, out],
        out_shape=[jax.ShapeDtypeStruct((N0, H), jnp.float32)] * 4,
    )(x0, x1, wr0, wo0, wr1, wo1, b0, b1)


def _tc2_body(a1_ref, r1_ref, a0_ref, r0_ref, wrel_ref, wroot_ref, b2_ref,
              u1_ref, root2_ref):
    h_v1 = jnp.maximum(a1_ref[...] + r1_ref[...], 0.0)
    h_v0 = jnp.maximum(a0_ref[...] + r0_ref[...], 0.0)
    pad = ((0, 0), (0, CP - C))
    wrel = jnp.pad(wrel_ref[...], pad)
    wroot = jnp.pad(wroot_ref[...], pad)
    b2 = jnp.pad(b2_ref[...], pad)
    u1_ref[...] = jnp.dot(h_v1, wrel,
                          preferred_element_type=jnp.float32).astype(jnp.bfloat16)
    root2_ref[...] = jnp.dot(h_v0, wroot,
                             preferred_element_type=jnp.float32) + b2


def _tc2(a1, r1, a0, r0, wrel, wroot, b2):
    g = N0 // _ROW_BLK
    rowb = pl.BlockSpec((_ROW_BLK, H), lambda i: (i, 0))
    w = pl.BlockSpec((H, C), lambda i: (0, 0))
    b = pl.BlockSpec((1, C), lambda i: (0, 0))
    out = pl.BlockSpec((_ROW_BLK, CP), lambda i: (i, 0))
    return pl.pallas_call(
        _tc2_body,
        grid=(g,),
        in_specs=[rowb, rowb, rowb, rowb, w, w, b],
        out_specs=[out, out],
        out_shape=[jax.ShapeDtypeStruct((N0, CP), jnp.bfloat16),
                   jax.ShapeDtypeStruct((N0, CP), jnp.float32)],
    )(a1, r1, a0, r0, wrel, wroot, b2)


def _tc3_body(p0_ref, p1_ref, root_ref, y_ref, loss_ref):
    i = pl.program_id(0)
    x = (p0_ref[...] + p1_ref[...]).astype(jnp.float32) + root_ref[...]
    col = lax.broadcasted_iota(jnp.int32, (_ROW_BLK, CP), 1)
    valid = col < C
    xm = jnp.where(valid, x, -1e30)
    m = jnp.max(xm, axis=1, keepdims=True)
    ex = jnp.where(valid, jnp.exp(x - m), 0.0)
    lse = m[:, 0] + jnp.log(jnp.sum(ex, axis=1))
    pick = jnp.sum(jnp.where(col == y_ref[...], x, 0.0), axis=1)
    part = (jnp.sum(lse - pick) * (1.0 / N0)).reshape(1, 1)

    @pl.when(i == 0)
    def _():
        loss_ref[...] = jnp.zeros_like(loss_ref)

    loss_ref[...] += part


def _tc3(p0, p1, root2, y2d):
    g = N0 // _ROW_BLK
    full = pl.BlockSpec((_ROW_BLK, CP), lambda i: (i, 0))
    return pl.pallas_call(
        _tc3_body,
        grid=(g,),
        in_specs=[full, full, full, pl.BlockSpec((_ROW_BLK, 1), lambda i: (i, 0))],
        out_specs=pl.BlockSpec((1, 1), lambda i: (0, 0)),
        out_shape=jax.ShapeDtypeStruct((1, 1), jnp.float32),
    )(p0, p1, root2, y2d)


def _edge_pipeline(z_hbm, idx_src, idx_dst, acc, bufs, gsem, ssem, nch,
                   lead):
    nb = len(bufs)
    d = nb - lead
    for b in range(lead):
        pltpu.async_copy(z_hbm.at[idx_src.at[b]], bufs[b], gsem)

    @pl.loop(0, nch, step=nb)
    def _slots(g):
        for b in range(nb):
            ch = g + b
            buf = bufs[b]
            pltpu.make_async_copy(z_hbm.at[idx_src.at[ch]], buf, gsem).wait()
            pltpu.async_copy(buf, acc.at[idx_dst.at[ch]], ssem, add=True)

            @pl.when(ch >= d)
            def _():
                pltpu.make_async_copy(bufs[(b - d) % nb],
                                      acc.at[idx_dst.at[ch]], ssem).wait()

            @pl.when(ch + lead < nch)
            def _():
                pltpu.async_copy(z_hbm.at[idx_src.at[ch + lead]],
                                 bufs[(b + lead) % nb], gsem)

    for b in range(d):
        pltpu.make_async_copy(bufs[b], acc.at[idx_dst.at[0]], ssem).wait()


_NCH1 = E // NS // K


def _sc1_body(z0_hbm, ei0_hbm, z1_hbm, ei1_hbm,
              zeros_hbm, out0_hbm, out1_hbm,
              acc, idx_src, idx_dst, b0, b1, b2, b3, b4, b5, b6, b7, b8, b9,
              gsem, ssem):
    bufs = (b0, b1, b2, b3, b4, b5, b6, b7, b8, b9)
    c = lax.axis_index("c")
    s = lax.axis_index("s")
    pltpu.sync_copy(zeros_hbm, acc.at[pl.ds(s * ROWS_PER_TILE, ROWS_PER_TILE)])
    plsc.subcore_barrier()

    for p in range(2):
        @pl.when(c == 0)
        def _():
            pltpu.sync_copy(ei0_hbm.at[0, s, p], idx_src)
            pltpu.sync_copy(ei0_hbm.at[1, s, p], idx_dst)
            _edge_pipeline(z0_hbm, idx_src, idx_dst, acc, bufs, gsem, ssem,
                           _NCH1 // 2, lead=6)

        @pl.when(c == 1)
        def _():
            pltpu.sync_copy(ei1_hbm.at[0, s, p], idx_src)
            pltpu.sync_copy(ei1_hbm.at[1, s, p], idx_dst)
            _edge_pipeline(z1_hbm, idx_src, idx_dst, acc, bufs, gsem, ssem,
                           _NCH1 // 2, lead=6)

    plsc.subcore_barrier()
    rows = pl.ds(s * ROWS_PER_TILE, ROWS_PER_TILE)

    @pl.when(c == 0)
    def _():
        pltpu.sync_copy(acc.at[rows], out0_hbm.at[rows])

    @pl.when(c == 1)
    def _():
        pltpu.sync_copy(acc.at[rows], out1_hbm.at[rows])


def _sc1(z0, ei0, z1, ei1, zeros_h):
    mesh = plsc.VectorSubcoreMesh(core_axis_name="c", subcore_axis_name="s")
    return pl.kernel(
        _sc1_body,
        compiler_params=pltpu.CompilerParams(use_tc_tiling_on_sc=False),
        out_type=[jax.ShapeDtypeStruct((NPAD, H), jnp.float32),
                  jax.ShapeDtypeStruct((NPAD, H), jnp.float32)],
        mesh=mesh,
        scratch_types=[
            pltpu.VMEM_SHARED((NPAD, H), jnp.float32),
            pltpu.VMEM((_NCH1 // 2, K), jnp.int32),
            pltpu.VMEM((_NCH1 // 2, K), jnp.int32),
            pltpu.VMEM((K, H), jnp.float32),
            pltpu.VMEM((K, H), jnp.float32),
            pltpu.VMEM((K, H), jnp.float32),
            pltpu.VMEM((K, H), jnp.float32),
            pltpu.VMEM((K, H), jnp.float32),
            pltpu.VMEM((K, H), jnp.float32),
            pltpu.VMEM((K, H), jnp.float32),
            pltpu.VMEM((K, H), jnp.float32),
            pltpu.VMEM((K, H), jnp.float32),
            pltpu.VMEM((K, H), jnp.float32),
            pltpu.SemaphoreType.DMA,
            pltpu.SemaphoreType.DMA,
        ],
    )(z0, ei0, z1, ei1, zeros_h)


_NCH2 = E // (NC * NS) // K


def _sc2_body(u1_hbm, ei_hbm, zeros_hbm, out0_hbm, out1_hbm,
              acc, idx_src, idx_dst, b0, b1, b2, b3, b4, gsem, ssem):
    bufs = (b0, b1, b2, b3, b4)
    c = lax.axis_index("c")
    s = lax.axis_index("s")
    pltpu.sync_copy(zeros_hbm, acc.at[pl.ds(s * ROWS_PER_TILE, ROWS_PER_TILE)])
    plsc.subcore_barrier()

    for p in range(2):
        pltpu.sync_copy(ei_hbm.at[0, c, s, p], idx_src)
        pltpu.sync_copy(ei_hbm.at[1, c, s, p], idx_dst)
        _edge_pipeline(u1_hbm, idx_src, idx_dst, acc, bufs, gsem, ssem,
                       _NCH2 // 2, lead=3)

    plsc.subcore_barrier()
    rows = pl.ds(s * ROWS_PER_TILE, ROWS_PER_TILE)

    @pl.when(c == 0)
    def _():
        pltpu.sync_copy(acc.at[rows], out0_hbm.at[rows])

    @pl.when(c == 1)
    def _():
        pltpu.sync_copy(acc.at[rows], out1_hbm.at[rows])


def _sc2(u1, ei1, zeros_h):
    mesh = plsc.VectorSubcoreMesh(core_axis_name="c", subcore_axis_name="s")
    return pl.kernel(
        _sc2_body,
        compiler_params=pltpu.CompilerParams(use_tc_tiling_on_sc=False),
        out_type=[jax.ShapeDtypeStruct((NPAD, CP), jnp.bfloat16),
                  jax.ShapeDtypeStruct((NPAD, CP), jnp.bfloat16)],
        mesh=mesh,
        scratch_types=[
            pltpu.VMEM_SHARED((NPAD, CP), jnp.bfloat16),
            pltpu.VMEM((_NCH2 // 2, K), jnp.int32),
            pltpu.VMEM((_NCH2 // 2, K), jnp.int32),
            pltpu.VMEM((K, CP), jnp.bfloat16),
            pltpu.VMEM((K, CP), jnp.bfloat16),
            pltpu.VMEM((K, CP), jnp.bfloat16),
            pltpu.VMEM((K, CP), jnp.bfloat16),
            pltpu.VMEM((K, CP), jnp.bfloat16),
            pltpu.SemaphoreType.DMA,
            pltpu.SemaphoreType.DMA,
        ],
    )(u1, ei1, zeros_h)


def kernel(x0, x1, edge_index_0, edge_index_1, y,
           W1_rel_0, b1_0, W1_root_0,
           W1_rel_1, b1_1, W1_root_1,
           W2_rel_0, b2_0, W2_root_0,
           W2_rel_1, b2_1, W2_root_1):
    ei0 = edge_index_0.astype(jnp.int32).reshape(2, NS, 2, _NCH1 // 2, K)
    ei1 = edge_index_1.astype(jnp.int32)
    ei1_l1 = ei1.reshape(2, NS, 2, _NCH1 // 2, K)
    ei1_l2 = ei1.reshape(2, NC, NS, 2, _NCH2 // 2, K)

    z0, z1, r1, r0 = _tc1(x0, x1, W1_rel_0, W1_root_0, W1_rel_1, W1_root_1,
                          b1_0.reshape(1, H), b1_1.reshape(1, H))

    zeros32 = jnp.zeros((ROWS_PER_TILE, H), jnp.float32)
    agg_v1, agg_v0 = _sc1(z0, ei0, z1, ei1_l1, zeros32)

    u1, root2 = _tc2(agg_v1, r1, agg_v0, r0, W2_rel_1, W2_root_1,
                     b2_1.reshape(1, C))

    zeros16 = jnp.zeros((ROWS_PER_TILE, CP), jnp.bfloat16)
    p0, p1 = _sc2(u1, ei1_l2, zeros16)

    y2d = y.astype(jnp.int32).reshape(N0, 1)
    loss = _tc3(p0, p1, root2, y2d)
    return loss[0, 0]


# speedup vs baseline: 1.3045x; 1.0182x over previous
import jax
import jax.numpy as jnp
from jax import lax
from jax.experimental import pallas as pl
from jax.experimental.pallas import tpu as pltpu
from jax.experimental.pallas import tpu_sc as plsc

N0 = 20000
N1 = 20000
E = 640000
D = 64
H = 32
C = 10
CP = 16

NC = 2
NS = 16
NPAD = 20480
ROWS_PER_TILE = NPAD // NS
K = 80

_ROW_BLK = 2000


def _tc1_body(x0_ref, x1_ref, wr0_ref, wo0_ref, wr1_ref, wo1_ref,
              b0_ref, b1_ref, z0_ref, z1_ref, r1_ref, r0_ref):
    x0 = x0_ref[...]
    x1 = x1_ref[...]
    z0_ref[...] = jnp.dot(x0, wr0_ref[...], preferred_element_type=jnp.float32)
    z1_ref[...] = jnp.dot(x1, wr1_ref[...], preferred_element_type=jnp.float32)
    r1_ref[...] = jnp.dot(x1, wo0_ref[...],
                          preferred_element_type=jnp.float32) + b0_ref[...]
    r0_ref[...] = jnp.dot(x0, wo1_ref[...],
                          preferred_element_type=jnp.float32) + b1_ref[...]


def _tc1(x0, x1, wr0, wo0, wr1, wo1, b0, b1):
    g = N0 // _ROW_BLK
    row = pl.BlockSpec((_ROW_BLK, D), lambda i: (i, 0))
    w = pl.BlockSpec((D, H), lambda i: (0, 0))
    b = pl.BlockSpec((1, H), lambda i: (0, 0))
    out = pl.BlockSpec((_ROW_BLK, H), lambda i: (i, 0))
    return pl.pallas_call(
        _tc1_body,
        grid=(g,),
        in_specs=[row, row, w, w, w, w, b, b],
        out_specs=[out, out, out, out],
        out_shape=[jax.ShapeDtypeStruct((N0, H), jnp.float32)] * 4,
    )(x0, x1, wr0, wo0, wr1, wo1, b0, b1)


def _tc2_body(a1_ref, r1_ref, a0_ref, r0_ref, wrel_ref, wroot_ref, b2_ref,
              u1_ref, root2_ref):
    h_v1 = jnp.maximum(a1_ref[...] + r1_ref[...], 0.0)
    h_v0 = jnp.maximum(a0_ref[...] + r0_ref[...], 0.0)
    pad = ((0, 0), (0, CP - C))
    wrel = jnp.pad(wrel_ref[...], pad)
    wroot = jnp.pad(wroot_ref[...], pad)
    b2 = jnp.pad(b2_ref[...], pad)
    u1_ref[...] = jnp.dot(h_v1, wrel,
                          preferred_element_type=jnp.float32).astype(jnp.bfloat16)
    root2_ref[...] = jnp.dot(h_v0, wroot,
                             preferred_element_type=jnp.float32) + b2


def _tc2(a1, r1, a0, r0, wrel, wroot, b2):
    g = N0 // _ROW_BLK
    rowb = pl.BlockSpec((_ROW_BLK, H), lambda i: (i, 0))
    w = pl.BlockSpec((H, C), lambda i: (0, 0))
    b = pl.BlockSpec((1, C), lambda i: (0, 0))
    out = pl.BlockSpec((_ROW_BLK, CP), lambda i: (i, 0))
    return pl.pallas_call(
        _tc2_body,
        grid=(g,),
        in_specs=[rowb, rowb, rowb, rowb, w, w, b],
        out_specs=[out, out],
        out_shape=[jax.ShapeDtypeStruct((N0, CP), jnp.bfloat16),
                   jax.ShapeDtypeStruct((N0, CP), jnp.float32)],
    )(a1, r1, a0, r0, wrel, wroot, b2)


def _tc3_body(p0_ref, p1_ref, root_ref, y_ref, loss_ref):
    i = pl.program_id(0)
    x = (p0_ref[...] + p1_ref[...]).astype(jnp.float32) + root_ref[...]
    col = lax.broadcasted_iota(jnp.int32, (_ROW_BLK, CP), 1)
    valid = col < C
    xm = jnp.where(valid, x, -1e30)
    m = jnp.max(xm, axis=1, keepdims=True)
    ex = jnp.where(valid, jnp.exp(x - m), 0.0)
    lse = m[:, 0] + jnp.log(jnp.sum(ex, axis=1))
    pick = jnp.sum(jnp.where(col == y_ref[...], x, 0.0), axis=1)
    part = (jnp.sum(lse - pick) * (1.0 / N0)).reshape(1, 1)

    @pl.when(i == 0)
    def _():
        loss_ref[...] = jnp.zeros_like(loss_ref)

    loss_ref[...] += part


def _tc3(p0, p1, root2, y2d):
    g = N0 // _ROW_BLK
    full = pl.BlockSpec((_ROW_BLK, CP), lambda i: (i, 0))
    return pl.pallas_call(
        _tc3_body,
        grid=(g,),
        in_specs=[full, full, full, pl.BlockSpec((_ROW_BLK, 1), lambda i: (i, 0))],
        out_specs=pl.BlockSpec((1, 1), lambda i: (0, 0)),
        out_shape=jax.ShapeDtypeStruct((1, 1), jnp.float32),
    )(p0, p1, root2, y2d)


def _edge_pipeline(z_hbm, idx_src, idx_dst, acc, bufs, gsem, ssem, nch,
                   lead):
    nb = len(bufs)
    d = nb - lead
    for b in range(lead):
        pltpu.async_copy(z_hbm.at[idx_src.at[b]], bufs[b], gsem)

    @pl.loop(0, nch, step=nb)
    def _slots(g):
        for b in range(nb):
            ch = g + b
            buf = bufs[b]
            pltpu.make_async_copy(z_hbm.at[idx_src.at[ch]], buf, gsem).wait()
            pltpu.async_copy(buf, acc.at[idx_dst.at[ch]], ssem, add=True)

            @pl.when(ch >= d)
            def _():
                pltpu.make_async_copy(bufs[(b - d) % nb],
                                      acc.at[idx_dst.at[ch]], ssem).wait()

            @pl.when(ch + lead < nch)
            def _():
                pltpu.async_copy(z_hbm.at[idx_src.at[ch + lead]],
                                 bufs[(b + lead) % nb], gsem)

    for b in range(d):
        pltpu.make_async_copy(bufs[b], acc.at[idx_dst.at[0]], ssem).wait()


_NCH1 = E // NS // K


def _sc1_body(z0_hbm, ei0_hbm, z1_hbm, ei1_hbm,
              zeros_hbm, out0_hbm, out1_hbm,
              acc, idx_src, idx_dst, b0, b1, b2, b3, b4, b5, b6, b7, b8, b9,
              gsem, ssem):
    bufs = (b0, b1, b2, b3, b4, b5, b6, b7, b8, b9)
    c = lax.axis_index("c")
    s = lax.axis_index("s")
    pltpu.sync_copy(zeros_hbm, acc.at[pl.ds(s * ROWS_PER_TILE, ROWS_PER_TILE)])
    plsc.subcore_barrier()

    for p in range(2):
        @pl.when(c == 0)
        def _():
            pltpu.sync_copy(ei0_hbm.at[0, s, p], idx_src)
            pltpu.sync_copy(ei0_hbm.at[1, s, p], idx_dst)
            _edge_pipeline(z0_hbm, idx_src, idx_dst, acc, bufs, gsem, ssem,
                           _NCH1 // 2, lead=7)

        @pl.when(c == 1)
        def _():
            pltpu.sync_copy(ei1_hbm.at[0, s, p], idx_src)
            pltpu.sync_copy(ei1_hbm.at[1, s, p], idx_dst)
            _edge_pipeline(z1_hbm, idx_src, idx_dst, acc, bufs, gsem, ssem,
                           _NCH1 // 2, lead=7)

    plsc.subcore_barrier()
    rows = pl.ds(s * ROWS_PER_TILE, ROWS_PER_TILE)

    @pl.when(c == 0)
    def _():
        pltpu.sync_copy(acc.at[rows], out0_hbm.at[rows])

    @pl.when(c == 1)
    def _():
        pltpu.sync_copy(acc.at[rows], out1_hbm.at[rows])


def _sc1(z0, ei0, z1, ei1, zeros_h):
    mesh = plsc.VectorSubcoreMesh(core_axis_name="c", subcore_axis_name="s")
    return pl.kernel(
        _sc1_body,
        compiler_params=pltpu.CompilerParams(use_tc_tiling_on_sc=False),
        out_type=[jax.ShapeDtypeStruct((NPAD, H), jnp.float32),
                  jax.ShapeDtypeStruct((NPAD, H), jnp.float32)],
        mesh=mesh,
        scratch_types=[
            pltpu.VMEM_SHARED((NPAD, H), jnp.float32),
            pltpu.VMEM((_NCH1 // 2, K), jnp.int32),
            pltpu.VMEM((_NCH1 // 2, K), jnp.int32),
            pltpu.VMEM((K, H), jnp.float32),
            pltpu.VMEM((K, H), jnp.float32),
            pltpu.VMEM((K, H), jnp.float32),
            pltpu.VMEM((K, H), jnp.float32),
            pltpu.VMEM((K, H), jnp.float32),
            pltpu.VMEM((K, H), jnp.float32),
            pltpu.VMEM((K, H), jnp.float32),
            pltpu.VMEM((K, H), jnp.float32),
            pltpu.VMEM((K, H), jnp.float32),
            pltpu.VMEM((K, H), jnp.float32),
            pltpu.SemaphoreType.DMA,
            pltpu.SemaphoreType.DMA,
        ],
    )(z0, ei0, z1, ei1, zeros_h)


_NCH2 = E // (NC * NS) // K


def _sc2_body(u1_hbm, ei_hbm, zeros_hbm, out0_hbm, out1_hbm,
              acc, idx_src, idx_dst, b0, b1, b2, b3, b4, gsem, ssem):
    bufs = (b0, b1, b2, b3, b4)
    c = lax.axis_index("c")
    s = lax.axis_index("s")
    pltpu.sync_copy(zeros_hbm, acc.at[pl.ds(s * ROWS_PER_TILE, ROWS_PER_TILE)])
    plsc.subcore_barrier()

    for p in range(2):
        pltpu.sync_copy(ei_hbm.at[0, c, s, p], idx_src)
        pltpu.sync_copy(ei_hbm.at[1, c, s, p], idx_dst)
        _edge_pipeline(u1_hbm, idx_src, idx_dst, acc, bufs, gsem, ssem,
                       _NCH2 // 2, lead=4)

    plsc.subcore_barrier()
    rows = pl.ds(s * ROWS_PER_TILE, ROWS_PER_TILE)

    @pl.when(c == 0)
    def _():
        pltpu.sync_copy(acc.at[rows], out0_hbm.at[rows])

    @pl.when(c == 1)
    def _():
        pltpu.sync_copy(acc.at[rows], out1_hbm.at[rows])


def _sc2(u1, ei1, zeros_h):
    mesh = plsc.VectorSubcoreMesh(core_axis_name="c", subcore_axis_name="s")
    return pl.kernel(
        _sc2_body,
        compiler_params=pltpu.CompilerParams(use_tc_tiling_on_sc=False),
        out_type=[jax.ShapeDtypeStruct((NPAD, CP), jnp.bfloat16),
                  jax.ShapeDtypeStruct((NPAD, CP), jnp.bfloat16)],
        mesh=mesh,
        scratch_types=[
            pltpu.VMEM_SHARED((NPAD, CP), jnp.bfloat16),
            pltpu.VMEM((_NCH2 // 2, K), jnp.int32),
            pltpu.VMEM((_NCH2 // 2, K), jnp.int32),
            pltpu.VMEM((K, CP), jnp.bfloat16),
            pltpu.VMEM((K, CP), jnp.bfloat16),
            pltpu.VMEM((K, CP), jnp.bfloat16),
            pltpu.VMEM((K, CP), jnp.bfloat16),
            pltpu.VMEM((K, CP), jnp.bfloat16),
            pltpu.SemaphoreType.DMA,
            pltpu.SemaphoreType.DMA,
        ],
    )(u1, ei1, zeros_h)


def kernel(x0, x1, edge_index_0, edge_index_1, y,
           W1_rel_0, b1_0, W1_root_0,
           W1_rel_1, b1_1, W1_root_1,
           W2_rel_0, b2_0, W2_root_0,
           W2_rel_1, b2_1, W2_root_1):
    ei0 = edge_index_0.astype(jnp.int32).reshape(2, NS, 2, _NCH1 // 2, K)
    ei1 = edge_index_1.astype(jnp.int32)
    ei1_l1 = ei1.reshape(2, NS, 2, _NCH1 // 2, K)
    ei1_l2 = ei1.reshape(2, NC, NS, 2, _NCH2 // 2, K)

    z0, z1, r1, r0 = _tc1(x0, x1, W1_rel_0, W1_root_0, W1_rel_1, W1_root_1,
                          b1_0.reshape(1, H), b1_1.reshape(1, H))

    zeros32 = jnp.zeros((ROWS_PER_TILE, H), jnp.float32)
    agg_v1, agg_v0 = _sc1(z0, ei0, z1, ei1_l1, zeros32)

    u1, root2 = _tc2(agg_v1, r1, agg_v0, r0, W2_rel_1, W2_root_1,
                     b2_1.reshape(1, C))

    zeros16 = jnp.zeros((ROWS_PER_TILE, CP), jnp.bfloat16)
    p0, p1 = _sc2(u1, ei1_l2, zeros16)

    y2d = y.astype(jnp.int32).reshape(N0, 1)
    loss = _tc3(p0, p1, root2, y2d)
    return loss[0, 0]
